# Initial kernel scaffold; baseline (speedup 1.0000x reference)
import jax
import jax.numpy as jnp
from jax.experimental import pallas as pl


def kernel(M, x0, y0, x1, y1, wx, wy):
    raise NotImplementedError("write your pallas kernel here")


# speedup vs baseline: 36.0889x; 36.0889x over previous
import functools

import jax
import jax.numpy as jnp
from jax import lax
from jax.experimental import pallas as pl
from jax.experimental.pallas import tpu as pltpu
from jax.experimental.pallas import tpu_sc as plsc

N = 262144
Q = 32
ROWS = 256 * 256
NC, NS, L = 2, 16, 16
NW = NC * NS
PER_W = N // NW
C = 128
N_CHUNKS = PER_W // C


def _body(m_hbm, x0_hbm, y0_hbm, x1_hbm, y1_hbm, wx_hbm, wy_hbm, out_hbm,
          x0_v, y0_v, x1_v, y1_v, wx_v, wy_v,
          i00_v, i01_v, i10_v, i11_v,
          r00_v, r01_v, r10_v, r11_v, out_v, sem):
    wid = lax.axis_index("s") * NC + lax.axis_index("c")

    def chunk_body(c, carry):
        base = wid * PER_W + c * C

        pltpu.sync_copy(x0_hbm.at[pl.ds(base, C)], x0_v)
        pltpu.sync_copy(y0_hbm.at[pl.ds(base, C)], y0_v)
        pltpu.sync_copy(x1_hbm.at[pl.ds(base, C)], x1_v)
        pltpu.sync_copy(y1_hbm.at[pl.ds(base, C)], y1_v)
        pltpu.sync_copy(wx_hbm.at[pl.ds(base, C)], wx_v)
        pltpu.sync_copy(wy_hbm.at[pl.ds(base, C)], wy_v)

        def idx_body(k, carry):
            s = pl.ds(k * L, L)
            x0s = x0_v[s]
            y0s = y0_v[s]
            x1s = x1_v[s]
            y1s = y1_v[s]
            y0b = y0s * 256
            y1b = y1s * 256
            i00_v[s] = y0b + x0s
            i01_v[s] = y0b + x1s
            i10_v[s] = y1b + x0s
            i11_v[s] = y1b + x1s
            return carry

        lax.fori_loop(0, C // L, idx_body, 0)

        cp0 = pltpu.async_copy(m_hbm.at[i00_v], r00_v, sem)
        cp1 = pltpu.async_copy(m_hbm.at[i01_v], r01_v, sem)
        cp2 = pltpu.async_copy(m_hbm.at[i10_v], r10_v, sem)
        cp3 = pltpu.async_copy(m_hbm.at[i11_v], r11_v, sem)
        cp0.wait()
        cp1.wait()
        cp2.wait()
        cp3.wait()

        def blend_body(k, carry):
            s = pl.ds(k * L, L)
            wx = wx_v[s]
            wy = wy_v[s]
            w11 = wx * wy
            w01 = wx - w11
            w10 = wy - w11
            w00 = 1.0 - wx - wy + w11
            for p in range(L):
                pp = k * L + p

                def bc(wv):
                    return lax.broadcast(
                        lax.squeeze(lax.slice_in_dim(wv, p, p + 1), [0]), (L,))

                b00, b01, b10, b11 = bc(w00), bc(w01), bc(w10), bc(w11)
                for h in range(Q // L):
                    col = pl.ds(h * L, L)
                    o = (r00_v[pp, col] * b00 + r01_v[pp, col] * b01
                         + r10_v[pp, col] * b10 + r11_v[pp, col] * b11)
                    out_v[pp, col] = o
            return carry

        lax.fori_loop(0, C // L, blend_body, 0)

        pltpu.sync_copy(out_v, out_hbm.at[pl.ds(base, C)])
        return carry

    lax.fori_loop(0, N_CHUNKS, chunk_body, 0)


def kernel(M, x0, y0, x1, y1, wx, wy):
    m_flat = M.reshape(ROWS, Q)
    wx_f = wx.reshape(N)
    wy_f = wy.reshape(N)

    mesh = plsc.VectorSubcoreMesh(core_axis_name="c", subcore_axis_name="s")
    run = functools.partial(
        pl.kernel,
        out_type=jax.ShapeDtypeStruct((N, Q), jnp.float32),
        mesh=mesh,
        scratch_types=[
            pltpu.VMEM((C,), jnp.int32),
            pltpu.VMEM((C,), jnp.int32),
            pltpu.VMEM((C,), jnp.int32),
            pltpu.VMEM((C,), jnp.int32),
            pltpu.VMEM((C,), jnp.float32),
            pltpu.VMEM((C,), jnp.float32),
            pltpu.VMEM((C,), jnp.int32),
            pltpu.VMEM((C,), jnp.int32),
            pltpu.VMEM((C,), jnp.int32),
            pltpu.VMEM((C,), jnp.int32),
            pltpu.VMEM((C, Q), jnp.float32),
            pltpu.VMEM((C, Q), jnp.float32),
            pltpu.VMEM((C, Q), jnp.float32),
            pltpu.VMEM((C, Q), jnp.float32),
            pltpu.VMEM((C, Q), jnp.float32),
            pltpu.SemaphoreType.DMA,
        ],
        compiler_params=pltpu.CompilerParams(
            use_tc_tiling_on_sc=False,
        ),
    )(_body)
    return run(m_flat, x0, y0, x1, y1, wx_f, wy_f)


# speedup vs baseline: 68.7623x; 1.9054x over previous
import functools

import jax
import jax.numpy as jnp
from jax import lax
from jax.experimental import pallas as pl
from jax.experimental.pallas import tpu as pltpu
from jax.experimental.pallas import tpu_sc as plsc

N = 262144
Q = 32
ROWS = 256 * 256
NC, NS, L = 2, 16, 16
NW = NC * NS
PER_W = N // NW
C = 128
N_CHUNKS = PER_W // C
NPAIR = N_CHUNKS // 2


def _make_buf():
    return dict(
        i00=pltpu.VMEM((C,), jnp.int32),
        i01=pltpu.VMEM((C,), jnp.int32),
        i10=pltpu.VMEM((C,), jnp.int32),
        i11=pltpu.VMEM((C,), jnp.int32),
        r00=pltpu.VMEM((C, Q), jnp.float32),
        r01=pltpu.VMEM((C, Q), jnp.float32),
        r10=pltpu.VMEM((C, Q), jnp.float32),
        r11=pltpu.VMEM((C, Q), jnp.float32),
        out=pltpu.VMEM((C, Q), jnp.float32),
        semG=pltpu.SemaphoreType.DMA,
        semO=pltpu.SemaphoreType.DMA,
    )


def _body(m_hbm, x0_hbm, y0_hbm, x1_hbm, y1_hbm, wx_hbm, wy_hbm, out_hbm,
          x0_v, y0_v, x1_v, y1_v, wx_v, wy_v, bufs, semI):
    wid = lax.axis_index("s") * NC + lax.axis_index("c")
    wbase = wid * PER_W

    ins = (
        (x0_hbm, x0_v), (y0_hbm, y0_v), (x1_hbm, x1_v), (y1_hbm, y1_v),
        (wx_hbm, wx_v), (wy_hbm, wy_v),
    )
    cps = [pltpu.async_copy(h.at[pl.ds(wbase, PER_W)], v, semI) for h, v in ins]
    for cp in cps:
        cp.wait()

    def prep(c, b):
        def idx_body(k, carry):
            src = pl.ds(c * C + k * L, L)
            dst = pl.ds(k * L, L)
            x0s = x0_v[src]
            y0s = y0_v[src]
            x1s = x1_v[src]
            y1s = y1_v[src]
            y0b = y0s * 256
            y1b = y1s * 256
            b["i00"][dst] = y0b + x0s
            b["i01"][dst] = y0b + x1s
            b["i10"][dst] = y1b + x0s
            b["i11"][dst] = y1b + x1s
            return carry

        lax.fori_loop(0, C // L, idx_body, 0)
        pltpu.async_copy(m_hbm.at[b["i00"]], b["r00"], b["semG"])
        pltpu.async_copy(m_hbm.at[b["i01"]], b["r01"], b["semG"])
        pltpu.async_copy(m_hbm.at[b["i10"]], b["r10"], b["semG"])
        pltpu.async_copy(m_hbm.at[b["i11"]], b["r11"], b["semG"])

    def finish(c, b, kk):
        @pl.when(kk > 0)
        def _():
            pltpu.make_async_copy(
                b["out"], out_hbm.at[pl.ds(wbase + c * C, C)], b["semO"]).wait()

        pltpu.make_async_copy(m_hbm.at[b["i00"]], b["r00"], b["semG"]).wait()
        pltpu.make_async_copy(m_hbm.at[b["i01"]], b["r01"], b["semG"]).wait()
        pltpu.make_async_copy(m_hbm.at[b["i10"]], b["r10"], b["semG"]).wait()
        pltpu.make_async_copy(m_hbm.at[b["i11"]], b["r11"], b["semG"]).wait()

        def blend_body(k, carry):
            s = pl.ds(c * C + k * L, L)
            wx = wx_v[s]
            wy = wy_v[s]
            w11 = wx * wy
            w01 = wx - w11
            w10 = wy - w11
            w00 = 1.0 - wx - wy + w11
            for p in range(L):
                pp = k * L + p

                def bc(wv):
                    return lax.broadcast(
                        lax.squeeze(lax.slice_in_dim(wv, p, p + 1), [0]), (L,))

                b00, b01, b10, b11 = bc(w00), bc(w01), bc(w10), bc(w11)
                for h in range(Q // L):
                    col = pl.ds(h * L, L)
                    o = (b["r00"][pp, col] * b00 + b["r01"][pp, col] * b01
                         + b["r10"][pp, col] * b10 + b["r11"][pp, col] * b11)
                    b["out"][pp, col] = o
            return carry

        lax.fori_loop(0, C // L, blend_body, 0)
        pltpu.async_copy(b["out"], out_hbm.at[pl.ds(wbase + c * C, C)],
                         b["semO"])

    buf_a, buf_b = bufs
    prep(0, buf_a)

    def pair_body(kk, carry):
        c0 = 2 * kk
        prep(c0 + 1, buf_b)
        finish(c0, buf_a, kk)

        @pl.when(kk < NPAIR - 1)
        def _():
            prep(c0 + 2, buf_a)

        finish(c0 + 1, buf_b, kk)
        return carry

    lax.fori_loop(0, NPAIR, pair_body, 0)

    last = N_CHUNKS - 2
    pltpu.make_async_copy(
        buf_a["out"], out_hbm.at[pl.ds(wbase + last * C, C)],
        buf_a["semO"]).wait()
    pltpu.make_async_copy(
        buf_b["out"], out_hbm.at[pl.ds(wbase + (last + 1) * C, C)],
        buf_b["semO"]).wait()


def kernel(M, x0, y0, x1, y1, wx, wy):
    m_flat = M.reshape(ROWS, Q)
    wx_f = wx.reshape(N)
    wy_f = wy.reshape(N)

    mesh = plsc.VectorSubcoreMesh(core_axis_name="c", subcore_axis_name="s")
    run = functools.partial(
        pl.kernel,
        out_type=jax.ShapeDtypeStruct((N, Q), jnp.float32),
        mesh=mesh,
        scratch_types=[
            pltpu.VMEM((PER_W,), jnp.int32),
            pltpu.VMEM((PER_W,), jnp.int32),
            pltpu.VMEM((PER_W,), jnp.int32),
            pltpu.VMEM((PER_W,), jnp.int32),
            pltpu.VMEM((PER_W,), jnp.float32),
            pltpu.VMEM((PER_W,), jnp.float32),
            [_make_buf(), _make_buf()],
            pltpu.SemaphoreType.DMA,
        ],
        compiler_params=pltpu.CompilerParams(
            use_tc_tiling_on_sc=False,
        ),
    )(_body)
    return run(m_flat, x0, y0, x1, y1, wx_f, wy_f)
